# baseline (speedup 1.0000x reference)
import dataclasses
import functools

import jax
import jax.numpy as jnp
from jax import lax
from jax.experimental import pallas as pl
from jax.experimental.pallas import tpu as pltpu
from jax.experimental.pallas import tpu_sc as plsc

S = 2048
D = 1024
F = 2048
E = 8
K = 2
TM = 256
GMAX = 24
PMAX = GMAX * TM
NW = 32
RPW = PMAX // NW
CH = 64
CHD = 32
NEG = -1e30


def _sc_compiler_params():
    cp = pltpu.CompilerParams()
    if "needs_layout_passes" in pltpu.CompilerParams.__dataclass_fields__:
        cp = dataclasses.replace(cp, needs_layout_passes=False)
    return cp


def _router_body(x_ref, wgt_ref, bgb_ref, d0_ref, d1_ref, w0_ref, w1_ref,
                 te_ref):
    x = x_ref[...]
    logits = jnp.dot(x, wgt_ref[...], preferred_element_type=jnp.float32)
    logits = logits + bgb_ref[0:1, :]

    ecols = lax.broadcasted_iota(jnp.int32, (S, E), 1)
    v0 = jnp.max(logits, axis=1, keepdims=True)
    e0 = jnp.min(jnp.where(logits == v0, ecols, E), axis=1, keepdims=True)
    oh0 = ecols == e0
    masked = jnp.where(oh0, NEG, logits)
    v1 = jnp.max(masked, axis=1, keepdims=True)
    e1 = jnp.min(jnp.where(masked == v1, ecols, E), axis=1, keepdims=True)
    oh1 = ecols == e1

    r = jnp.exp(v1 - v0)
    denom = 1.0 + r
    w0_ref[...] = 1.0 / denom
    w1_ref[...] = r / denom

    oh0f = oh0.astype(jnp.float32)
    oh1f = oh1.astype(jnp.float32)
    cnt0 = jnp.sum(oh0f, axis=0, keepdims=True)
    cnt1 = jnp.sum(oh1f, axis=0, keepdims=True)
    cnt = cnt0 + cnt1
    tiles = jnp.floor((cnt + (TM - 0.5)) * (1.0 / TM))
    erow = lax.broadcasted_iota(jnp.int32, (E, E), 0)
    ecol2 = lax.broadcasted_iota(jnp.int32, (E, E), 1)
    strict_lower = (erow < ecol2).astype(jnp.float32)
    tile_start = jnp.dot(tiles, strict_lower,
                         preferred_element_type=jnp.float32)
    poff = tile_start * TM

    tri = (lax.broadcasted_iota(jnp.int32, (1, S), 1)
           < lax.broadcasted_iota(jnp.int32, (S, 1), 0)).astype(jnp.bfloat16)
    oh01 = jnp.concatenate([oh0f, oh1f], axis=1).astype(jnp.bfloat16)
    s01 = jnp.dot(tri, oh01, preferred_element_type=jnp.float32)
    s0 = s01[:, :E]
    s1 = s01[:, E:]
    rank0 = jnp.sum(s0 * oh0f, axis=1, keepdims=True)
    rank1 = (jnp.sum(s1 * oh1f, axis=1, keepdims=True)
             + jnp.sum(cnt0 * oh1f, axis=1, keepdims=True))
    dest0 = jnp.sum(poff * oh0f, axis=1, keepdims=True) + rank0
    dest1 = jnp.sum(poff * oh1f, axis=1, keepdims=True) + rank1
    d0_ref[...] = dest0.astype(jnp.int32)
    d1_ref[...] = dest1.astype(jnp.int32)

    total = jnp.sum(tiles)
    gi = lax.broadcasted_iota(jnp.int32, (GMAX + 8, 1), 0).astype(jnp.float32)
    gic = jnp.minimum(gi, total - 1.0)
    te = jnp.sum((tile_start <= gic).astype(jnp.float32), axis=1,
                 keepdims=True) - 1.0
    te = jnp.where(gi == GMAX, total, te)
    te_ref[...] = te.astype(jnp.int32)


def _router(x2d, wgt, bgb):
    return pl.pallas_call(
        _router_body,
        out_shape=[
            jax.ShapeDtypeStruct((S, 1), jnp.int32),
            jax.ShapeDtypeStruct((S, 1), jnp.int32),
            jax.ShapeDtypeStruct((S, 1), jnp.float32),
            jax.ShapeDtypeStruct((S, 1), jnp.float32),
            jax.ShapeDtypeStruct((GMAX + 8, 1), jnp.int32),
        ],
    )(x2d, wgt, bgb)


def _dispatch_body(x_hbm, dest_hbm, xs_hbm, idx_v, rows_a, rows_b, sem_a,
                   sem_b):
    wid = lax.axis_index("s") * 2 + lax.axis_index("c")
    nch = (K * S) // (NW * CHD)
    bufs, sems = (rows_a, rows_b), (sem_a, sem_b)
    pltpu.sync_copy(dest_hbm.at[pl.ds(wid * nch, nch)], idx_v)
    descs = []
    for c in range(nch):
        b = c & 1
        if c >= 2:
            descs[c - 2].wait()
        a_base = (wid * nch + c) * CHD
        tok = pl.multiple_of(a_base & (S - 1), CHD)
        pltpu.sync_copy(x_hbm.at[pl.ds(tok, CHD)], bufs[b])
        descs.append(pltpu.async_copy(bufs[b], xs_hbm.at[idx_v.at[c]],
                                      sems[b]))
    descs[-2].wait()
    descs[-1].wait()


def _dispatch(x2d, dest2d):
    mesh = plsc.VectorSubcoreMesh(core_axis_name="c", subcore_axis_name="s")
    nch = (K * S) // (NW * CHD)
    kern = pl.kernel(
        _dispatch_body,
        out_type=jax.ShapeDtypeStruct((PMAX, D), jnp.float32),
        mesh=mesh,
        scratch_types=[
            pltpu.VMEM((nch, CHD), jnp.int32),
            pltpu.VMEM((CHD, D), jnp.float32),
            pltpu.VMEM((CHD, D), jnp.float32),
            pltpu.SemaphoreType.DMA,
            pltpu.SemaphoreType.DMA,
        ],
        compiler_params=_sc_compiler_params(),
    )
    return kern(x2d, dest2d)


def _ffn_body(te_ref, xs_ref, w1_ref, b1_ref, w2_ref, b2_ref, ys_ref,
              w1b_ref, w2b_ref):
    i = pl.program_id(0)
    total = te_ref[GMAX]
    changed = jnp.logical_or(i == 0, te_ref[i] != te_ref[jnp.maximum(i - 1, 0)])

    @pl.when(jnp.logical_and(changed, i < total))
    def _():
        w1b_ref[...] = w1_ref[0].astype(jnp.bfloat16)
        w2b_ref[...] = w2_ref[0].astype(jnp.bfloat16)

    @pl.when(i < total)
    def _():
        xb = xs_ref[...].astype(jnp.bfloat16)
        h = jnp.dot(xb, w1b_ref[...], preferred_element_type=jnp.float32)
        h = jnp.maximum(h + b1_ref[0], 0.0).astype(jnp.bfloat16)
        y = jnp.dot(h, w2b_ref[...], preferred_element_type=jnp.float32)
        ys_ref[...] = y + b2_ref[0]


def _grouped_ffn(te, xs, W1, b1r, W2, b2r):
    def _imin(i, te):
        return jnp.minimum(i, te[GMAX] - 1)

    grid_spec = pltpu.PrefetchScalarGridSpec(
        num_scalar_prefetch=1,
        grid=(GMAX,),
        in_specs=[
            pl.BlockSpec((TM, D), lambda i, te: (_imin(i, te), 0)),
            pl.BlockSpec((1, D, F), lambda i, te: (te[i], 0, 0)),
            pl.BlockSpec((1, 1, F), lambda i, te: (te[i], 0, 0)),
            pl.BlockSpec((1, F, D), lambda i, te: (te[i], 0, 0)),
            pl.BlockSpec((1, 1, D), lambda i, te: (te[i], 0, 0)),
        ],
        out_specs=pl.BlockSpec((TM, D), lambda i, te: (_imin(i, te), 0)),
        scratch_shapes=[
            pltpu.VMEM((D, F), jnp.bfloat16),
            pltpu.VMEM((F, D), jnp.bfloat16),
        ],
    )
    return pl.pallas_call(
        _ffn_body,
        grid_spec=grid_spec,
        out_shape=jax.ShapeDtypeStruct((PMAX, D), jnp.float32),
    )(te, xs, W1, b1r, W2, b2r)


def _combine_body(ys_hbm, idx_hbm, gath_hbm, idx_v, rows_a, rows_b, sem_a,
                  sem_b):
    wid = lax.axis_index("s") * 2 + lax.axis_index("c")
    nch = (K * S) // (NW * CHD)
    bufs, sems = (rows_a, rows_b), (sem_a, sem_b)
    pltpu.sync_copy(idx_hbm.at[pl.ds(wid * nch, nch)], idx_v)
    descs = [pltpu.async_copy(ys_hbm.at[idx_v.at[c]], bufs[c], sems[c])
             for c in range(2)]
    for c in range(nch):
        b = c & 1
        descs[c].wait()
        pltpu.sync_copy(bufs[b],
                        gath_hbm.at[pl.ds((wid * nch + c) * CHD, CHD)])
        if c + 2 < nch:
            descs.append(pltpu.async_copy(ys_hbm.at[idx_v.at[c + 2]],
                                          bufs[b], sems[b]))


def _combine_gather(ys, idx2d):
    mesh = plsc.VectorSubcoreMesh(core_axis_name="c", subcore_axis_name="s")
    nch = (K * S) // (NW * CHD)
    kern = pl.kernel(
        _combine_body,
        out_type=jax.ShapeDtypeStruct((K * S, D), jnp.float32),
        mesh=mesh,
        scratch_types=[
            pltpu.VMEM((nch, CHD), jnp.int32),
            pltpu.VMEM((CHD, D), jnp.float32),
            pltpu.VMEM((CHD, D), jnp.float32),
            pltpu.SemaphoreType.DMA,
            pltpu.SemaphoreType.DMA,
        ],
        compiler_params=_sc_compiler_params(),
    )
    return kern(ys, idx2d)


def _wadd_body(g0_ref, g1_ref, w0_ref, w1_ref, o_ref):
    o_ref[...] = w0_ref[...] * g0_ref[...] + w1_ref[...] * g1_ref[...]


def _weighted_add(gath, w0, w1):
    nblk = S // TM
    return pl.pallas_call(
        _wadd_body,
        grid=(nblk,),
        in_specs=[
            pl.BlockSpec((TM, D), lambda i: (i, 0)),
            pl.BlockSpec((TM, D), lambda i: (i + nblk, 0)),
            pl.BlockSpec((TM, 1), lambda i: (i, 0)),
            pl.BlockSpec((TM, 1), lambda i: (i, 0)),
        ],
        out_specs=pl.BlockSpec((TM, D), lambda i: (i, 0)),
        out_shape=jax.ShapeDtypeStruct((S, D), jnp.float32),
    )(gath, gath, w0, w1)


def kernel(x, Wg, bg, bias, W1, b1, W2, b2):
    x2d = x.reshape(S, D)
    wgt = jnp.transpose(Wg)
    bgb = jnp.broadcast_to(bg + bias, (8, E))
    b1r = b1.reshape(E, 1, F)
    b2r = b2.reshape(E, 1, D)

    dest0, dest1, w0, w1, te = _router(x2d, wgt, bgb)
    dest = jnp.concatenate([dest0, dest1], axis=0)
    nchw = (K * S) // (NW * CHD)
    dest2d = dest.reshape(NW * nchw, CHD)
    xs = _dispatch(x2d, dest2d)
    ys = _grouped_ffn(te.reshape(GMAX + 8), xs, W1, b1r, W2, b2r)
    gath = _combine_gather(ys, dest2d)
    out = _weighted_add(gath, w0, w1)
    return out.reshape(S, 1, D)


# speedup vs baseline: 1.0108x; 1.0108x over previous
import dataclasses
import functools

import jax
import jax.numpy as jnp
from jax import lax
from jax.experimental import pallas as pl
from jax.experimental.pallas import tpu as pltpu
from jax.experimental.pallas import tpu_sc as plsc

S = 2048
D = 1024
F = 2048
E = 8
K = 2
TM = 256
GMAX = 24
PMAX = GMAX * TM
NW = 32
RPW = PMAX // NW
CH = 64
CHD = 32
NEG = -1e30


def _sc_compiler_params():
    cp = pltpu.CompilerParams()
    if "needs_layout_passes" in pltpu.CompilerParams.__dataclass_fields__:
        cp = dataclasses.replace(cp, needs_layout_passes=False)
    return cp


def _router_body(x_ref, wgt_ref, bgb_ref, d0_ref, d1_ref, w0_ref, w1_ref,
                 te_ref):
    x = x_ref[...]
    logits = jnp.dot(x, wgt_ref[...], preferred_element_type=jnp.float32)
    logits = logits + bgb_ref[0:1, :]

    ecols = lax.broadcasted_iota(jnp.int32, (S, E), 1)
    v0 = jnp.max(logits, axis=1, keepdims=True)
    e0 = jnp.min(jnp.where(logits == v0, ecols, E), axis=1, keepdims=True)
    oh0 = ecols == e0
    masked = jnp.where(oh0, NEG, logits)
    v1 = jnp.max(masked, axis=1, keepdims=True)
    e1 = jnp.min(jnp.where(masked == v1, ecols, E), axis=1, keepdims=True)
    oh1 = ecols == e1

    r = jnp.exp(v1 - v0)
    denom = 1.0 + r
    w0_ref[...] = 1.0 / denom
    w1_ref[...] = r / denom

    oh0f = oh0.astype(jnp.float32)
    oh1f = oh1.astype(jnp.float32)
    cnt0 = jnp.sum(oh0f, axis=0, keepdims=True)
    cnt1 = jnp.sum(oh1f, axis=0, keepdims=True)
    cnt = cnt0 + cnt1
    tiles = jnp.floor((cnt + (TM - 0.5)) * (1.0 / TM))
    erow = lax.broadcasted_iota(jnp.int32, (E, E), 0)
    ecol2 = lax.broadcasted_iota(jnp.int32, (E, E), 1)
    strict_lower = (erow < ecol2).astype(jnp.float32)
    tile_start = jnp.dot(tiles, strict_lower,
                         preferred_element_type=jnp.float32)
    poff = tile_start * TM

    tri = (lax.broadcasted_iota(jnp.int32, (1, S), 1)
           < lax.broadcasted_iota(jnp.int32, (S, 1), 0)).astype(jnp.bfloat16)
    oh01 = jnp.concatenate([oh0f, oh1f], axis=1).astype(jnp.bfloat16)
    s01 = jnp.dot(tri, oh01, preferred_element_type=jnp.float32)
    s0 = s01[:, :E]
    s1 = s01[:, E:]
    rank0 = jnp.sum(s0 * oh0f, axis=1, keepdims=True)
    rank1 = (jnp.sum(s1 * oh1f, axis=1, keepdims=True)
             + jnp.sum(cnt0 * oh1f, axis=1, keepdims=True))
    dest0 = jnp.sum(poff * oh0f, axis=1, keepdims=True) + rank0
    dest1 = jnp.sum(poff * oh1f, axis=1, keepdims=True) + rank1
    d0_ref[...] = dest0.astype(jnp.int32)
    d1_ref[...] = dest1.astype(jnp.int32)

    total = jnp.sum(tiles)
    gi = lax.broadcasted_iota(jnp.int32, (GMAX + 8, 1), 0).astype(jnp.float32)
    gic = jnp.minimum(gi, total - 1.0)
    te = jnp.sum((tile_start <= gic).astype(jnp.float32), axis=1,
                 keepdims=True) - 1.0
    te = jnp.where(gi == GMAX, total, te)
    te_ref[...] = te.astype(jnp.int32)


def _router(x2d, wgt, bgb):
    return pl.pallas_call(
        _router_body,
        out_shape=[
            jax.ShapeDtypeStruct((S, 1), jnp.int32),
            jax.ShapeDtypeStruct((S, 1), jnp.int32),
            jax.ShapeDtypeStruct((S, 1), jnp.float32),
            jax.ShapeDtypeStruct((S, 1), jnp.float32),
            jax.ShapeDtypeStruct((GMAX + 8, 1), jnp.int32),
        ],
    )(x2d, wgt, bgb)


def _dispatch_body(x_hbm, dest_hbm, xs_hbm, idx_v, rows_v, sem):
    wid = lax.axis_index("s") * 2 + lax.axis_index("c")
    nch = (K * S) // (NW * CH)
    pltpu.sync_copy(dest_hbm.at[pl.ds(wid * nch, nch)], idx_v)
    for c in range(nch):
        a_base = (wid * nch + c) * CH
        tok = pl.multiple_of(a_base & (S - 1), CH)
        pltpu.sync_copy(x_hbm.at[pl.ds(tok, CH)], rows_v)
        pltpu.async_copy(rows_v, xs_hbm.at[idx_v.at[c]], sem).wait()


def _dispatch(x2d, dest2d):
    mesh = plsc.VectorSubcoreMesh(core_axis_name="c", subcore_axis_name="s")
    nch = (K * S) // (NW * CH)
    kern = pl.kernel(
        _dispatch_body,
        out_type=jax.ShapeDtypeStruct((PMAX, D), jnp.float32),
        mesh=mesh,
        scratch_types=[
            pltpu.VMEM((nch, CH), jnp.int32),
            pltpu.VMEM((CH, D), jnp.float32),
            pltpu.SemaphoreType.DMA,
        ],
        compiler_params=_sc_compiler_params(),
    )
    return kern(x2d, dest2d)


def _ffn_body(te_ref, xs_ref, w1_ref, b1_ref, w2_ref, b2_ref, ys_ref,
              w1b_ref, w2b_ref):
    i = pl.program_id(0)
    total = te_ref[GMAX]
    changed = jnp.logical_or(i == 0, te_ref[i] != te_ref[jnp.maximum(i - 1, 0)])

    @pl.when(jnp.logical_and(changed, i < total))
    def _():
        w1b_ref[...] = w1_ref[0].astype(jnp.bfloat16)
        w2b_ref[...] = w2_ref[0].astype(jnp.bfloat16)

    @pl.when(i < total)
    def _():
        xb = xs_ref[...].astype(jnp.bfloat16)
        h = jnp.dot(xb, w1b_ref[...], preferred_element_type=jnp.float32)
        h = jnp.maximum(h + b1_ref[0], 0.0).astype(jnp.bfloat16)
        y = jnp.dot(h, w2b_ref[...], preferred_element_type=jnp.float32)
        ys_ref[...] = y + b2_ref[0]


def _grouped_ffn(te, xs, W1, b1r, W2, b2r):
    def _imin(i, te):
        return jnp.minimum(i, te[GMAX] - 1)

    grid_spec = pltpu.PrefetchScalarGridSpec(
        num_scalar_prefetch=1,
        grid=(GMAX,),
        in_specs=[
            pl.BlockSpec((TM, D), lambda i, te: (_imin(i, te), 0)),
            pl.BlockSpec((1, D, F), lambda i, te: (te[i], 0, 0)),
            pl.BlockSpec((1, 1, F), lambda i, te: (te[i], 0, 0)),
            pl.BlockSpec((1, F, D), lambda i, te: (te[i], 0, 0)),
            pl.BlockSpec((1, 1, D), lambda i, te: (te[i], 0, 0)),
        ],
        out_specs=pl.BlockSpec((TM, D), lambda i, te: (_imin(i, te), 0)),
        scratch_shapes=[
            pltpu.VMEM((D, F), jnp.bfloat16),
            pltpu.VMEM((F, D), jnp.bfloat16),
        ],
    )
    return pl.pallas_call(
        _ffn_body,
        grid_spec=grid_spec,
        out_shape=jax.ShapeDtypeStruct((PMAX, D), jnp.float32),
    )(te, xs, W1, b1r, W2, b2r)


def _combine_body(ys_hbm, idx_hbm, gath_hbm, idx_v, rows_v, sem):
    wid = lax.axis_index("s") * 2 + lax.axis_index("c")
    nch = (K * S) // (NW * CH)
    pltpu.sync_copy(idx_hbm.at[pl.ds(wid * nch, nch)], idx_v)
    for c in range(nch):
        pltpu.async_copy(ys_hbm.at[idx_v.at[c]], rows_v, sem).wait()
        pltpu.sync_copy(rows_v,
                        gath_hbm.at[pl.ds((wid * nch + c) * CH, CH)])


def _combine_gather(ys, idx2d):
    mesh = plsc.VectorSubcoreMesh(core_axis_name="c", subcore_axis_name="s")
    nch = (K * S) // (NW * CH)
    kern = pl.kernel(
        _combine_body,
        out_type=jax.ShapeDtypeStruct((K * S, D), jnp.float32),
        mesh=mesh,
        scratch_types=[
            pltpu.VMEM((nch, CH), jnp.int32),
            pltpu.VMEM((CH, D), jnp.float32),
            pltpu.SemaphoreType.DMA,
        ],
        compiler_params=_sc_compiler_params(),
    )
    return kern(ys, idx2d)


def _wadd_body(g0_ref, g1_ref, w0_ref, w1_ref, o_ref):
    o_ref[...] = w0_ref[...] * g0_ref[...] + w1_ref[...] * g1_ref[...]


def _weighted_add(gath, w0, w1):
    nblk = S // TM
    return pl.pallas_call(
        _wadd_body,
        grid=(nblk,),
        in_specs=[
            pl.BlockSpec((TM, D), lambda i: (i, 0)),
            pl.BlockSpec((TM, D), lambda i: (i + nblk, 0)),
            pl.BlockSpec((TM, 1), lambda i: (i, 0)),
            pl.BlockSpec((TM, 1), lambda i: (i, 0)),
        ],
        out_specs=pl.BlockSpec((TM, D), lambda i: (i, 0)),
        out_shape=jax.ShapeDtypeStruct((S, D), jnp.float32),
    )(gath, gath, w0, w1)


def kernel(x, Wg, bg, bias, W1, b1, W2, b2):
    x2d = x.reshape(S, D)
    wgt = jnp.transpose(Wg)
    bgb = jnp.broadcast_to(bg + bias, (8, E))
    b1r = b1.reshape(E, 1, F)
    b2r = b2.reshape(E, 1, D)

    dest0, dest1, w0, w1, te = _router(x2d, wgt, bgb)
    dest = jnp.concatenate([dest0, dest1], axis=0)
    nchw = (K * S) // (NW * CH)
    dest2d = dest.reshape(NW * nchw, CH)
    xs = _dispatch(x2d, dest2d)
    ys = _grouped_ffn(te.reshape(GMAX + 8), xs, W1, b1r, W2, b2r)
    gath = _combine_gather(ys, dest2d)
    out = _weighted_add(gath, w0, w1)
    return out.reshape(S, 1, D)


# speedup vs baseline: 1.1417x; 1.1296x over previous
import dataclasses
import functools

import jax
import jax.numpy as jnp
from jax import lax
from jax.experimental import pallas as pl
from jax.experimental.pallas import tpu as pltpu
from jax.experimental.pallas import tpu_sc as plsc

S = 2048
D = 1024
F = 2048
E = 8
K = 2
TM = 256
GMAX = 24
PMAX = GMAX * TM
NW = 32
RPW = PMAX // NW
CH = 64
CHD = 32
NEG = -1e30


def _sc_compiler_params():
    cp = pltpu.CompilerParams()
    if "needs_layout_passes" in pltpu.CompilerParams.__dataclass_fields__:
        cp = dataclasses.replace(cp, needs_layout_passes=False)
    return cp


def _router_body(x_ref, wgt_ref, bgb_ref, d0_ref, d1_ref, w0_ref, w1_ref,
                 te_ref):
    x = x_ref[...].reshape(S, D)
    logits = jnp.dot(x, wgt_ref[...], preferred_element_type=jnp.float32)
    logits = logits + bgb_ref[0:1, :]

    ecols = lax.broadcasted_iota(jnp.int32, (S, E), 1)
    v0 = jnp.max(logits, axis=1, keepdims=True)
    e0 = jnp.min(jnp.where(logits == v0, ecols, E), axis=1, keepdims=True)
    oh0 = ecols == e0
    masked = jnp.where(oh0, NEG, logits)
    v1 = jnp.max(masked, axis=1, keepdims=True)
    e1 = jnp.min(jnp.where(masked == v1, ecols, E), axis=1, keepdims=True)
    oh1 = ecols == e1

    r = jnp.exp(v1 - v0)
    denom = 1.0 + r
    w0_ref[...] = 1.0 / denom
    w1_ref[...] = r / denom

    oh0f = oh0.astype(jnp.float32)
    oh1f = oh1.astype(jnp.float32)
    cnt0 = jnp.sum(oh0f, axis=0, keepdims=True)
    cnt1 = jnp.sum(oh1f, axis=0, keepdims=True)
    cnt = cnt0 + cnt1
    tiles = jnp.floor((cnt + (TM - 0.5)) * (1.0 / TM))
    erow = lax.broadcasted_iota(jnp.int32, (E, E), 0)
    ecol2 = lax.broadcasted_iota(jnp.int32, (E, E), 1)
    strict_lower = (erow < ecol2).astype(jnp.float32)
    tile_start = jnp.dot(tiles, strict_lower,
                         preferred_element_type=jnp.float32)
    poff = tile_start * TM

    tri = (lax.broadcasted_iota(jnp.int32, (1, S), 1)
           < lax.broadcasted_iota(jnp.int32, (S, 1), 0)).astype(jnp.bfloat16)
    s0 = jnp.dot(tri, oh0f.astype(jnp.bfloat16),
                 preferred_element_type=jnp.float32)
    s1 = jnp.dot(tri, oh1f.astype(jnp.bfloat16),
                 preferred_element_type=jnp.float32)
    rank0 = jnp.sum(s0 * oh0f, axis=1, keepdims=True)
    rank1 = (jnp.sum(s1 * oh1f, axis=1, keepdims=True)
             + jnp.sum(cnt0 * oh1f, axis=1, keepdims=True))
    dest0 = jnp.sum(poff * oh0f, axis=1, keepdims=True) + rank0
    dest1 = jnp.sum(poff * oh1f, axis=1, keepdims=True) + rank1
    d0_ref[...] = dest0.astype(jnp.int32)
    d1_ref[...] = dest1.astype(jnp.int32)

    total = jnp.sum(tiles)
    gi = lax.broadcasted_iota(jnp.int32, (GMAX + 8, 1), 0).astype(jnp.float32)
    gic = jnp.minimum(gi, total - 1.0)
    te = jnp.sum((tile_start <= gic).astype(jnp.float32), axis=1,
                 keepdims=True) - 1.0
    te = jnp.where(gi == GMAX, total, te)
    te_ref[...] = te.astype(jnp.int32)


def _router(x2d, wgt, bgb):
    return pl.pallas_call(
        _router_body,
        out_shape=[
            jax.ShapeDtypeStruct((S, 1), jnp.int32),
            jax.ShapeDtypeStruct((S, 1), jnp.int32),
            jax.ShapeDtypeStruct((S, 1), jnp.float32),
            jax.ShapeDtypeStruct((S, 1), jnp.float32),
            jax.ShapeDtypeStruct((GMAX + 8, 1), jnp.int32),
        ],
    )(x2d, wgt, bgb)


def _dispatch_body(x_hbm, dest_hbm, xs_hbm, idx_v, rows_v, sem):
    wid = lax.axis_index("s") * 2 + lax.axis_index("c")
    nch = (K * S) // (NW * CH)
    pltpu.sync_copy(dest_hbm.at[pl.ds(wid * nch, nch)], idx_v)
    for c in range(nch):
        a_base = (wid * nch + c) * CH
        tok = pl.multiple_of(a_base & (S - 1), CH)
        pltpu.sync_copy(x_hbm.at[pl.ds(tok, CH)], rows_v)
        pltpu.async_copy(rows_v, xs_hbm.at[idx_v.at[c]], sem).wait()


def _dispatch(x3, dest2d):
    mesh = plsc.VectorSubcoreMesh(core_axis_name="c", subcore_axis_name="s")
    nch = (K * S) // (NW * CH)
    kern = pl.kernel(
        _dispatch_body,
        out_type=jax.ShapeDtypeStruct((PMAX, 8, D // 8), jnp.float32),
        mesh=mesh,
        scratch_types=[
            pltpu.VMEM((nch, CH), jnp.int32),
            pltpu.VMEM((CH, 8, D // 8), jnp.float32),
            pltpu.SemaphoreType.DMA,
        ],
        compiler_params=_sc_compiler_params(),
    )
    return kern(x3, dest2d)


def _ffn_body(te_ref, xs_ref, w1_ref, b1_ref, w2_ref, b2_ref, ys_ref,
              w1b_ref, w2b_ref):
    i = pl.program_id(0)
    total = te_ref[GMAX]
    changed = jnp.logical_or(i == 0, te_ref[i] != te_ref[jnp.maximum(i - 1, 0)])

    @pl.when(jnp.logical_and(changed, i < total))
    def _():
        w1b_ref[...] = w1_ref[0].astype(jnp.bfloat16)
        w2b_ref[...] = w2_ref[0].astype(jnp.bfloat16)

    @pl.when(i < total)
    def _():
        xb = xs_ref[...].reshape(TM, D).astype(jnp.bfloat16)
        h = jnp.dot(xb, w1b_ref[...], preferred_element_type=jnp.float32)
        h = jnp.maximum(h + b1_ref[0], 0.0).astype(jnp.bfloat16)
        y = jnp.dot(h, w2b_ref[...], preferred_element_type=jnp.float32)
        ys_ref[...] = y + b2_ref[0]


def _grouped_ffn(te, xs, W1, b1r, W2, b2r):
    def _imin(i, te):
        return jnp.minimum(i, te[GMAX] - 1)

    grid_spec = pltpu.PrefetchScalarGridSpec(
        num_scalar_prefetch=1,
        grid=(GMAX,),
        in_specs=[
            pl.BlockSpec((TM, 8, D // 8), lambda i, te: (_imin(i, te), 0, 0)),
            pl.BlockSpec((1, D, F), lambda i, te: (te[i], 0, 0)),
            pl.BlockSpec((1, 1, F), lambda i, te: (te[i], 0, 0)),
            pl.BlockSpec((1, F, D), lambda i, te: (te[i], 0, 0)),
            pl.BlockSpec((1, 1, D), lambda i, te: (te[i], 0, 0)),
        ],
        out_specs=pl.BlockSpec((TM, D), lambda i, te: (_imin(i, te), 0)),
        scratch_shapes=[
            pltpu.VMEM((D, F), jnp.bfloat16),
            pltpu.VMEM((F, D), jnp.bfloat16),
        ],
    )
    return pl.pallas_call(
        _ffn_body,
        grid_spec=grid_spec,
        out_shape=jax.ShapeDtypeStruct((PMAX, D), jnp.float32),
    )(te, xs, W1, b1r, W2, b2r)


def _combine_body(ys_hbm, idx_hbm, gath_hbm, idx_v, rows_v, sem):
    wid = lax.axis_index("s") * 2 + lax.axis_index("c")
    nch = (K * S) // (NW * CH)
    pltpu.sync_copy(idx_hbm.at[pl.ds(wid * nch, nch)], idx_v)
    for c in range(nch):
        pltpu.async_copy(ys_hbm.at[idx_v.at[c]], rows_v, sem).wait()
        pltpu.sync_copy(rows_v,
                        gath_hbm.at[pl.ds((wid * nch + c) * CH, CH)])


def _combine_gather(ys, idx2d):
    mesh = plsc.VectorSubcoreMesh(core_axis_name="c", subcore_axis_name="s")
    nch = (K * S) // (NW * CH)
    kern = pl.kernel(
        _combine_body,
        out_type=jax.ShapeDtypeStruct((K * S, D), jnp.float32),
        mesh=mesh,
        scratch_types=[
            pltpu.VMEM((nch, CH), jnp.int32),
            pltpu.VMEM((CH, D), jnp.float32),
            pltpu.SemaphoreType.DMA,
        ],
        compiler_params=_sc_compiler_params(),
    )
    return kern(ys, idx2d)


def _wadd_body(g0_ref, g1_ref, w0_ref, w1_ref, o_ref):
    res = w0_ref[...] * g0_ref[...] + w1_ref[...] * g1_ref[...]
    o_ref[...] = res.reshape(o_ref.shape)


def _weighted_add(gath, w0, w1):
    nblk = S // TM
    return pl.pallas_call(
        _wadd_body,
        grid=(nblk,),
        in_specs=[
            pl.BlockSpec((TM, D), lambda i: (i, 0)),
            pl.BlockSpec((TM, D), lambda i: (i + nblk, 0)),
            pl.BlockSpec((TM, 1), lambda i: (i, 0)),
            pl.BlockSpec((TM, 1), lambda i: (i, 0)),
        ],
        out_specs=pl.BlockSpec((TM, 8, 128), lambda i: (i, 0, 0)),
        out_shape=jax.ShapeDtypeStruct((S, 8, 128), jnp.float32),
    )(gath, gath, w0, w1)


def kernel(x, Wg, bg, bias, W1, b1, W2, b2):
    x2d = x.reshape(S, D)
    x3 = x.reshape(S, 8, D // 8)
    wgt = jnp.transpose(Wg)
    bgb = jnp.broadcast_to(bg + bias, (8, E))
    b1r = b1.reshape(E, 1, F)
    b2r = b2.reshape(E, 1, D)

    dest0, dest1, w0, w1, te = _router(x3, wgt, bgb)
    dest = jnp.concatenate([dest0, dest1], axis=0)
    nchw = (K * S) // (NW * CH)
    dest2d = dest.reshape(NW * nchw, CH)
    xs = _dispatch(x3, dest2d)
    ys = _grouped_ffn(te.reshape(GMAX + 8), xs, W1, b1r, W2, b2r)
    gath = _combine_gather(ys, dest2d)
    out = _weighted_add(gath, w0, w1)
    return out.reshape(S, 1, D)


# speedup vs baseline: 1.1465x; 1.0042x over previous
import dataclasses
import functools

import jax
import jax.numpy as jnp
from jax import lax
from jax.experimental import pallas as pl
from jax.experimental.pallas import tpu as pltpu
from jax.experimental.pallas import tpu_sc as plsc

S = 2048
D = 1024
F = 2048
E = 8
K = 2
TM = 256
GMAX = 24
PMAX = GMAX * TM
NW = 32
RPW = PMAX // NW
CH = 64
CHD = 32
NEG = -1e30


def _sc_compiler_params():
    cp = pltpu.CompilerParams()
    if "needs_layout_passes" in pltpu.CompilerParams.__dataclass_fields__:
        cp = dataclasses.replace(cp, needs_layout_passes=False)
    return cp


def _router_body(x_ref, wgt_ref, bgb_ref, d01_ref, w0_ref, w1_ref,
                 te_ref):
    x = x_ref[...].reshape(S, D)
    logits = jnp.dot(x, wgt_ref[...], preferred_element_type=jnp.float32)
    logits = logits + bgb_ref[0:1, :]

    ecols = lax.broadcasted_iota(jnp.int32, (S, E), 1)
    v0 = jnp.max(logits, axis=1, keepdims=True)
    e0 = jnp.min(jnp.where(logits == v0, ecols, E), axis=1, keepdims=True)
    oh0 = ecols == e0
    masked = jnp.where(oh0, NEG, logits)
    v1 = jnp.max(masked, axis=1, keepdims=True)
    e1 = jnp.min(jnp.where(masked == v1, ecols, E), axis=1, keepdims=True)
    oh1 = ecols == e1

    r = jnp.exp(v1 - v0)
    denom = 1.0 + r
    w0_ref[...] = 1.0 / denom
    w1_ref[...] = r / denom

    oh0f = oh0.astype(jnp.float32)
    oh1f = oh1.astype(jnp.float32)
    cnt0 = jnp.sum(oh0f, axis=0, keepdims=True)
    cnt1 = jnp.sum(oh1f, axis=0, keepdims=True)
    cnt = cnt0 + cnt1
    tiles = jnp.floor((cnt + (TM - 0.5)) * (1.0 / TM))
    erow = lax.broadcasted_iota(jnp.int32, (E, E), 0)
    ecol2 = lax.broadcasted_iota(jnp.int32, (E, E), 1)
    strict_lower = (erow < ecol2).astype(jnp.float32)
    tile_start = jnp.dot(tiles, strict_lower,
                         preferred_element_type=jnp.float32)
    poff = tile_start * TM

    tri = (lax.broadcasted_iota(jnp.int32, (1, S), 1)
           < lax.broadcasted_iota(jnp.int32, (S, 1), 0)).astype(jnp.bfloat16)
    s0 = jnp.dot(tri, oh0f.astype(jnp.bfloat16),
                 preferred_element_type=jnp.float32)
    s1 = jnp.dot(tri, oh1f.astype(jnp.bfloat16),
                 preferred_element_type=jnp.float32)
    rank0 = jnp.sum(s0 * oh0f, axis=1, keepdims=True)
    rank1 = (jnp.sum(s1 * oh1f, axis=1, keepdims=True)
             + jnp.sum(cnt0 * oh1f, axis=1, keepdims=True))
    dest0 = jnp.sum(poff * oh0f, axis=1, keepdims=True) + rank0
    dest1 = jnp.sum(poff * oh1f, axis=1, keepdims=True) + rank1
    d01_ref[0:S, :] = dest0.astype(jnp.int32)
    d01_ref[S:, :] = dest1.astype(jnp.int32)

    total = jnp.sum(tiles)
    gi = lax.broadcasted_iota(jnp.int32, (GMAX + 8, 1), 0).astype(jnp.float32)
    gic = jnp.minimum(gi, total - 1.0)
    te = jnp.sum((tile_start <= gic).astype(jnp.float32), axis=1,
                 keepdims=True) - 1.0
    te = jnp.where(gi == GMAX, total, te)
    te_ref[...] = te.astype(jnp.int32)


def _router(x2d, wgt, bgb):
    return pl.pallas_call(
        _router_body,
        out_shape=[
            jax.ShapeDtypeStruct((K * S, 1), jnp.int32),
            jax.ShapeDtypeStruct((S, 1), jnp.float32),
            jax.ShapeDtypeStruct((S, 1), jnp.float32),
            jax.ShapeDtypeStruct((GMAX + 8, 1), jnp.int32),
        ],
    )(x2d, wgt, bgb)


def _dispatch_body(x_hbm, dest_hbm, xs_hbm, idx_v, rows_v, sem):
    wid = lax.axis_index("s") * 2 + lax.axis_index("c")
    nch = (K * S) // (NW * CH)
    pltpu.sync_copy(dest_hbm.at[pl.ds(wid * nch, nch)], idx_v)
    for c in range(nch):
        a_base = (wid * nch + c) * CH
        tok = pl.multiple_of(a_base & (S - 1), CH)
        pltpu.sync_copy(x_hbm.at[pl.ds(tok, CH)], rows_v)
        pltpu.async_copy(rows_v, xs_hbm.at[idx_v.at[c]], sem).wait()


def _dispatch(x3, dest2d):
    mesh = plsc.VectorSubcoreMesh(core_axis_name="c", subcore_axis_name="s")
    nch = (K * S) // (NW * CH)
    kern = pl.kernel(
        _dispatch_body,
        out_type=jax.ShapeDtypeStruct((PMAX, 8, D // 8), jnp.float32),
        mesh=mesh,
        scratch_types=[
            pltpu.VMEM((nch, CH), jnp.int32),
            pltpu.VMEM((CH, 8, D // 8), jnp.float32),
            pltpu.SemaphoreType.DMA,
        ],
        compiler_params=_sc_compiler_params(),
    )
    return kern(x3, dest2d)


def _ffn_body(te_ref, xs_ref, w1_ref, b1_ref, w2_ref, b2_ref, ys_ref,
              w1b_ref, w2b_ref):
    i = pl.program_id(0)
    total = te_ref[GMAX]
    changed = jnp.logical_or(i == 0, te_ref[i] != te_ref[jnp.maximum(i - 1, 0)])

    @pl.when(jnp.logical_and(changed, i < total))
    def _():
        w1b_ref[...] = w1_ref[0].astype(jnp.bfloat16)
        w2b_ref[...] = w2_ref[0].astype(jnp.bfloat16)

    @pl.when(i < total)
    def _():
        xb = xs_ref[...].reshape(TM, D).astype(jnp.bfloat16)
        h = jnp.dot(xb, w1b_ref[...], preferred_element_type=jnp.float32)
        h = jnp.maximum(h + b1_ref[0], 0.0).astype(jnp.bfloat16)
        y = jnp.dot(h, w2b_ref[...], preferred_element_type=jnp.float32)
        ys_ref[...] = y + b2_ref[0]


def _grouped_ffn(te, xs, W1, b1r, W2, b2r):
    def _imin(i, te):
        return jnp.minimum(i, te[GMAX] - 1)

    grid_spec = pltpu.PrefetchScalarGridSpec(
        num_scalar_prefetch=1,
        grid=(GMAX,),
        in_specs=[
            pl.BlockSpec((TM, 8, D // 8), lambda i, te: (_imin(i, te), 0, 0)),
            pl.BlockSpec((1, D, F), lambda i, te: (te[i], 0, 0)),
            pl.BlockSpec((1, 1, F), lambda i, te: (te[i], 0, 0)),
            pl.BlockSpec((1, F, D), lambda i, te: (te[i], 0, 0)),
            pl.BlockSpec((1, 1, D), lambda i, te: (te[i], 0, 0)),
        ],
        out_specs=pl.BlockSpec((TM, D), lambda i, te: (_imin(i, te), 0)),
        scratch_shapes=[
            pltpu.VMEM((D, F), jnp.bfloat16),
            pltpu.VMEM((F, D), jnp.bfloat16),
        ],
    )
    return pl.pallas_call(
        _ffn_body,
        grid_spec=grid_spec,
        out_shape=jax.ShapeDtypeStruct((PMAX, D), jnp.float32),
    )(te, xs, W1, b1r, W2, b2r)


def _combine_body(ys_hbm, idx_hbm, gath_hbm, idx_v, rows_v, sem):
    wid = lax.axis_index("s") * 2 + lax.axis_index("c")
    nch = (K * S) // (NW * CH)
    pltpu.sync_copy(idx_hbm.at[pl.ds(wid * nch, nch)], idx_v)
    for c in range(nch):
        pltpu.async_copy(ys_hbm.at[idx_v.at[c]], rows_v, sem).wait()
        pltpu.sync_copy(rows_v,
                        gath_hbm.at[pl.ds((wid * nch + c) * CH, CH)])


def _combine_gather(ys, idx2d):
    mesh = plsc.VectorSubcoreMesh(core_axis_name="c", subcore_axis_name="s")
    nch = (K * S) // (NW * CH)
    kern = pl.kernel(
        _combine_body,
        out_type=jax.ShapeDtypeStruct((K * S, D), jnp.float32),
        mesh=mesh,
        scratch_types=[
            pltpu.VMEM((nch, CH), jnp.int32),
            pltpu.VMEM((CH, D), jnp.float32),
            pltpu.SemaphoreType.DMA,
        ],
        compiler_params=_sc_compiler_params(),
    )
    return kern(ys, idx2d)


def _wadd_body(g0_ref, g1_ref, w0_ref, w1_ref, o_ref):
    res = w0_ref[...] * g0_ref[...] + w1_ref[...] * g1_ref[...]
    o_ref[...] = res.reshape(o_ref.shape)


def _weighted_add(gath, w0, w1):
    nblk = S // TM
    return pl.pallas_call(
        _wadd_body,
        grid=(nblk,),
        in_specs=[
            pl.BlockSpec((TM, D), lambda i: (i, 0)),
            pl.BlockSpec((TM, D), lambda i: (i + nblk, 0)),
            pl.BlockSpec((TM, 1), lambda i: (i, 0)),
            pl.BlockSpec((TM, 1), lambda i: (i, 0)),
        ],
        out_specs=pl.BlockSpec((TM, 8, 128), lambda i: (i, 0, 0)),
        out_shape=jax.ShapeDtypeStruct((S, 8, 128), jnp.float32),
    )(gath, gath, w0, w1)


def kernel(x, Wg, bg, bias, W1, b1, W2, b2):
    x2d = x.reshape(S, D)
    x3 = x.reshape(S, 8, D // 8)
    wgt = jnp.transpose(Wg)
    bgb = jnp.broadcast_to(bg + bias, (8, E))
    b1r = b1.reshape(E, 1, F)
    b2r = b2.reshape(E, 1, D)

    dest, w0, w1, te = _router(x3, wgt, bgb)
    nchw = (K * S) // (NW * CH)
    dest2d = dest.reshape(NW * nchw, CH)
    xs = _dispatch(x3, dest2d)
    ys = _grouped_ffn(te.reshape(GMAX + 8), xs, W1, b1r, W2, b2r)
    gath = _combine_gather(ys, dest2d)
    out = _weighted_add(gath, w0, w1)
    return out.reshape(S, 1, D)
